# baseline (speedup 1.0000x reference)
import functools
import math
import numpy as np
import jax
import jax.numpy as jnp
from jax import lax
from jax.experimental import pallas as pl
from jax.experimental.pallas import tpu as pltpu
from jax.experimental.pallas import tpu_sc as plsc

_N = 512
_D = 128
_BI = 16
_NC, _NS, _L = 2, 16, 16
_NW = _NC * _NS
_RPW = _N // _NW

_ADJ = (0, 0, 0, 1, 1, 2009, 0)


def _build_pe_np():
    pe = np.zeros((_N, _D), np.float32)
    position = np.arange(0, _N, dtype=np.float32)[:, None]
    div = np.exp(np.arange(0, _D, 2).astype(np.float32) * (-math.log(10000.0) / _D))
    pe[:, 0::2] = np.sin(position * div)
    pe[:, 1::2] = np.cos(position * div)
    return pe


_PE_NP = _build_pe_np()


def _sc_lookup_body(cidx_hbm, hour_hbm, minute_hbm, second_hbm, day_hbm,
                    month_hbm, year_hbm, weekday_hbm, x_hbm,
                    idxv, acc, b0, b1, b2, b3, b4, b5, b6, sem):
    tables = (hour_hbm, minute_hbm, second_hbm, day_hbm, month_hbm,
              year_hbm, weekday_hbm)
    bufs = (b0, b1, b2, b3, b4, b5, b6)
    wid = lax.axis_index("s") * _NC + lax.axis_index("c")
    base = wid * _RPW
    pltpu.sync_copy(cidx_hbm.at[wid], idxv)
    copies = []
    for f in range(7):
        copies.append(
            pltpu.async_copy(tables[f].at[idxv.at[f]], bufs[f], sem))
    for c in copies:
        c.wait()
    for r in range(_RPW):
        for c in range(_D // _L):
            sl = pl.ds(c * _L, _L)
            v = bufs[0][r, sl]
            for f in range(1, 7):
                v = v + bufs[f][r, sl]
            acc[r, sl] = v
    pltpu.sync_copy(acc, x_hbm.at[pl.ds(base, _RPW)])


def _sc_lookup(cidx, hour_w, minute_w, second_w, day_w, month_w, year_w,
               weekday_w):
    mesh = plsc.VectorSubcoreMesh(core_axis_name="c", subcore_axis_name="s")
    return pl.kernel(
        _sc_lookup_body,
        out_type=jax.ShapeDtypeStruct((_N, _D), jnp.float32),
        mesh=mesh,
        scratch_types=[
            pltpu.VMEM((7, _L), jnp.int32),
            pltpu.VMEM((_RPW, _D), jnp.float32),
        ] + [pltpu.VMEM((_L, _D), jnp.float32) for _ in range(7)] + [
            pltpu.SemaphoreType.DMA,
        ],
    )(cidx, hour_w, minute_w, second_w, day_w, month_w, year_w, weekday_w)


def _bcast_body(x_ref, pe_ref, out_ref):
    out_ref[...] = x_ref[...][None, :, :] + pe_ref[...][:, None, :]


def kernel(time_features, hour_w, minute_w, second_w, day_w, month_w,
           year_w, weekday_w):
    pe = jnp.asarray(_PE_NP)
    adj = jnp.array(_ADJ, jnp.int32)
    cidx = (time_features.astype(jnp.int32) - adj[None, :]).reshape(
        _NW, _RPW, 7).transpose(0, 2, 1)
    x = _sc_lookup(cidx, hour_w, minute_w,
                   second_w, day_w, month_w, year_w, weekday_w)
    return pl.pallas_call(
        _bcast_body,
        grid=(_N // _BI,),
        in_specs=[
            pl.BlockSpec((_N, _D), lambda i: (0, 0)),
            pl.BlockSpec((_BI, _D), lambda i: (i, 0)),
        ],
        out_specs=pl.BlockSpec((_BI, _N, _D), lambda i: (i, 0, 0)),
        out_shape=jax.ShapeDtypeStruct((_N, _N, _D), jnp.float32),
    )(x, pe)


# speedup vs baseline: 1.6542x; 1.6542x over previous
import math
import numpy as np
import jax
import jax.numpy as jnp
from jax.experimental import pallas as pl
from jax.experimental.pallas import tpu as pltpu

_N = 512
_D = 128
_T_PAD = 256
_BI = 16

_BANDS = (
    (0, 0),
    (32, 0),
    (96, 0),
    (160, 1),
    (192, 1),
    (208, 2009),
    (216, 0),
)


def _build_pe_np():
    pe = np.zeros((_N, _D), np.float32)
    position = np.arange(0, _N, dtype=np.float32)[:, None]
    div = np.exp(np.arange(0, _D, 2).astype(np.float32) * (-math.log(10000.0) / _D))
    pe[:, 0::2] = np.sin(position * div)
    pe[:, 1::2] = np.cos(position * div)
    return pe


_PE_NP = _build_pe_np()


def _body(tf_ref, hour_ref, minute_ref, second_ref, day_ref, month_ref,
          year_ref, weekday_ref, pe_ref, out_ref, x_ref, w_ref):
    @pl.when(pl.program_id(0) == 0)
    def _():
        w_ref[...] = jnp.zeros((_T_PAD, _D), jnp.float32)
        tables = (hour_ref, minute_ref, second_ref, day_ref, month_ref,
                  year_ref, weekday_ref)
        for (off, _), t in zip(_BANDS, tables):
            w_ref[pl.ds(off, t.shape[0]), :] = t[...]
        idx = tf_ref[...]
        iota = jax.lax.broadcasted_iota(jnp.int32, (1, _T_PAD), 1)
        counts = jnp.zeros((_N, _T_PAD), jnp.float32)
        for k, (off, adj) in enumerate(_BANDS):
            counts += (idx[:, k:k + 1] == iota - (off - adj)).astype(
                jnp.float32)
        x_ref[...] = jnp.dot(counts, w_ref[...],
                             preferred_element_type=jnp.float32)
    i = pl.program_id(0)
    pe_blk = pe_ref[pl.ds(i * _BI, _BI), :]
    out_ref[...] = x_ref[...][None, :, :] + pe_blk[:, None, :]


def kernel(time_features, hour_w, minute_w, second_w, day_w, month_w,
           year_w, weekday_w):
    pe = jnp.asarray(_PE_NP)
    full = lambda shape: pl.BlockSpec(shape, lambda i: tuple(0 for _ in shape))
    return pl.pallas_call(
        _body,
        grid=(_N // _BI,),
        in_specs=[
            full((_N, 7)),
            full((24, _D)), full((60, _D)), full((60, _D)), full((31, _D)),
            full((12, _D)), full((3, _D)), full((7, _D)),
            full((_N, _D)),
        ],
        out_specs=pl.BlockSpec((_BI, _N, _D), lambda i: (i, 0, 0)),
        out_shape=jax.ShapeDtypeStruct((_N, _N, _D), jnp.float32),
        scratch_shapes=[pltpu.VMEM((_N, _D), jnp.float32),
                        pltpu.VMEM((_T_PAD, _D), jnp.float32)],
    )(time_features.astype(jnp.int32), hour_w, minute_w, second_w, day_w,
      month_w, year_w, weekday_w, pe)
